# Initial kernel scaffold; baseline (speedup 1.0000x reference)
import jax
import jax.numpy as jnp
from jax.experimental import pallas as pl


def kernel(node_tokens, edge_index, emb, sage_Wih, sage_Whh, sage_bih, sage_bhh, sage_selfW, sage_selfb, sage_neighW, sage_neighb, gat_W, gat_attn_l, gat_attn_r, gat_b, readout_W, readout_b):
    raise NotImplementedError("write your pallas kernel here")


# speedup vs baseline: 1.5845x; 1.5845x over previous
import functools

import jax
import jax.numpy as jnp
from jax import lax
from jax.experimental import pallas as pl
from jax.experimental.pallas import tpu as pltpu
from jax.experimental.pallas import tpu_sc as plsc

N_NODES = 10000
N_EDGES = 160000
VOCAB = 119
H = 500
HEADS = 5
NEG_SLOPE = 0.2

NP = 10240
HP = 512
G4 = 4 * HP
ZW = HEADS * HP
HW = 128

NW = 32
BPW = NP // NW


def _sc_gather(table, srcs, pos, D, C, two_level):
    nch = BPW // C
    mesh = plsc.VectorSubcoreMesh(core_axis_name="c", subcore_axis_name="s")

    @functools.partial(
        pl.kernel,
        mesh=mesh,
        out_type=jax.ShapeDtypeStruct((NP, D), jnp.float32),
        scratch_types=[
            pltpu.VMEM((nch, C), jnp.int32),
            pltpu.VMEM((C,), jnp.int32),
            pltpu.VMEM((C, D), jnp.float32),
            pltpu.SemaphoreType.DMA,
        ],
    )
    def k(table_hbm, srcs_hbm, pos_hbm, out_hbm, pos_v, idx_v, rows_v, sem):
        wid = lax.axis_index("s") * 2 + lax.axis_index("c")
        base = wid * BPW
        pltpu.sync_copy(pos_hbm.at[wid], pos_v)

        def body(j, carry):
            if two_level:
                pltpu.async_copy(srcs_hbm.at[pos_v.at[j]], idx_v, sem).wait()
                pltpu.async_copy(table_hbm.at[idx_v], rows_v, sem).wait()
            else:
                pltpu.async_copy(table_hbm.at[pos_v.at[j]], rows_v, sem).wait()
            pltpu.sync_copy(rows_v, out_hbm.at[pl.ds(base + j * C, C)])
            return carry

        lax.fori_loop(0, nch, body, 0)

    return k(table, srcs, pos)


def _mm_bias(x, w, b):
    BN = 512
    K, M = w.shape

    def body(x_r, w_r, b_r, o_r):
        o_r[...] = jnp.dot(x_r[...], w_r[...],
                           preferred_element_type=jnp.float32) + b_r[...]

    return pl.pallas_call(
        body,
        grid=(NP // BN,),
        in_specs=[
            pl.BlockSpec((BN, K), lambda i: (i, 0)),
            pl.BlockSpec((K, M), lambda i: (0, 0)),
            pl.BlockSpec((1, M), lambda i: (0, 0)),
        ],
        out_specs=pl.BlockSpec((BN, M), lambda i: (i, 0)),
        out_shape=jax.ShapeDtypeStruct((NP, M), jnp.float32),
    )(x, w, b)


def _lstm_step(xg, h, c, whh, maskf):
    BN = 512

    def body(xg_r, h_r, c_r, w_r, m_r, h2_r, c2_r):
        gates = xg_r[...] + jnp.dot(h_r[...], w_r[...],
                                    preferred_element_type=jnp.float32)
        gi = jax.nn.sigmoid(gates[:, 0 * HP:1 * HP])
        gf = jax.nn.sigmoid(gates[:, 1 * HP:2 * HP])
        gg = jnp.tanh(gates[:, 2 * HP:3 * HP])
        go = jax.nn.sigmoid(gates[:, 3 * HP:4 * HP])
        c_new = gf * c_r[...] + gi * gg
        h_new = go * jnp.tanh(c_new)
        sel = m_r[...] > 0.5
        c2_r[...] = jnp.where(sel, c_new, c_r[...])
        h2_r[...] = jnp.where(sel, h_new, h_r[...])

    return pl.pallas_call(
        body,
        grid=(NP // BN,),
        in_specs=[
            pl.BlockSpec((BN, G4), lambda i: (i, 0)),
            pl.BlockSpec((BN, HP), lambda i: (i, 0)),
            pl.BlockSpec((BN, HP), lambda i: (i, 0)),
            pl.BlockSpec((HP, G4), lambda i: (0, 0)),
            pl.BlockSpec((BN, 1), lambda i: (i, 0)),
        ],
        out_specs=[
            pl.BlockSpec((BN, HP), lambda i: (i, 0)),
            pl.BlockSpec((BN, HP), lambda i: (i, 0)),
        ],
        out_shape=[
            jax.ShapeDtypeStruct((NP, HP), jnp.float32),
            jax.ShapeDtypeStruct((NP, HP), jnp.float32),
        ],
    )(xg, h, c, whh, maskf)


def _combine(x, hn, wself, wneigh, b):
    BN = 512

    def body(x_r, h_r, ws_r, wn_r, b_r, o_r):
        acc = jnp.dot(x_r[...], ws_r[...], preferred_element_type=jnp.float32)
        acc = acc + jnp.dot(h_r[...], wn_r[...],
                            preferred_element_type=jnp.float32)
        o_r[...] = jnp.maximum(acc + b_r[...], 0.0)

    return pl.pallas_call(
        body,
        grid=(NP // BN,),
        in_specs=[
            pl.BlockSpec((BN, HP), lambda i: (i, 0)),
            pl.BlockSpec((BN, HP), lambda i: (i, 0)),
            pl.BlockSpec((HP, HP), lambda i: (0, 0)),
            pl.BlockSpec((HP, HP), lambda i: (0, 0)),
            pl.BlockSpec((1, HP), lambda i: (0, 0)),
        ],
        out_specs=pl.BlockSpec((BN, HP), lambda i: (i, 0)),
        out_shape=jax.ShapeDtypeStruct((NP, HP), jnp.float32),
    )(x, hn, wself, wneigh, b)


def _zel(x, wz, al, ar):
    BN = 512

    def body(x_r, wz_r, al_r, ar_r, z_r, el_r, er_r):
        z = jnp.dot(x_r[...], wz_r[...], preferred_element_type=jnp.float32)
        z_r[...] = z
        el_r[...] = jnp.dot(z, al_r[...], preferred_element_type=jnp.float32)
        er_r[...] = jnp.dot(z, ar_r[...], preferred_element_type=jnp.float32)

    return pl.pallas_call(
        body,
        grid=(NP // BN,),
        in_specs=[
            pl.BlockSpec((BN, HP), lambda i: (i, 0)),
            pl.BlockSpec((HP, ZW), lambda i: (0, 0)),
            pl.BlockSpec((ZW, HW), lambda i: (0, 0)),
            pl.BlockSpec((ZW, HW), lambda i: (0, 0)),
        ],
        out_specs=[
            pl.BlockSpec((BN, ZW), lambda i: (i, 0)),
            pl.BlockSpec((BN, HW), lambda i: (i, 0)),
            pl.BlockSpec((BN, HW), lambda i: (i, 0)),
        ],
        out_shape=[
            jax.ShapeDtypeStruct((NP, ZW), jnp.float32),
            jax.ShapeDtypeStruct((NP, HW), jnp.float32),
            jax.ShapeDtypeStruct((NP, HW), jnp.float32),
        ],
    )(x, wz, al, ar)


def _leaky(v):
    return jnp.where(v >= 0.0, v, NEG_SLOPE * v)


def _stats_step(gel, er, m, s, maskf):
    def body(gel_r, er_r, m_r, s_r, mk_r, m2_r, s2_r):
        e = _leaky(gel_r[...] + er_r[...])
        m_old = m_r[...]
        m_new = jnp.maximum(m_old, e)
        s_new = s_r[...] * jnp.exp(m_old - m_new) + jnp.exp(e - m_new)
        sel = mk_r[...] > 0.5
        m2_r[...] = jnp.where(sel, m_new, m_old)
        s2_r[...] = jnp.where(sel, s_new, s_r[...])

    BN = 2048
    return pl.pallas_call(
        body,
        grid=(NP // BN,),
        in_specs=[pl.BlockSpec((BN, HW), lambda i: (i, 0))] * 4
        + [pl.BlockSpec((BN, 1), lambda i: (i, 0))],
        out_specs=[pl.BlockSpec((BN, HW), lambda i: (i, 0))] * 2,
        out_shape=[jax.ShapeDtypeStruct((NP, HW), jnp.float32)] * 2,
    )(gel, er, m, s, maskf)


def _msg_step(acc, zg, gel, er, m, s, maskf, spread):
    BN = 512

    def body(a_r, zg_r, gel_r, er_r, m_r, s_r, mk_r, sp_r, o_r):
        e = _leaky(gel_r[...] + er_r[...])
        alpha = jnp.exp(e - m_r[...]) / jnp.maximum(s_r[...], 1e-9)
        alpha = jnp.where(mk_r[...] > 0.5, alpha, 0.0)
        aw = jnp.dot(alpha, sp_r[...], preferred_element_type=jnp.float32)
        o_r[...] = a_r[...] + aw * zg_r[...]

    return pl.pallas_call(
        body,
        grid=(NP // BN,),
        in_specs=[
            pl.BlockSpec((BN, ZW), lambda i: (i, 0)),
            pl.BlockSpec((BN, ZW), lambda i: (i, 0)),
            pl.BlockSpec((BN, HW), lambda i: (i, 0)),
            pl.BlockSpec((BN, HW), lambda i: (i, 0)),
            pl.BlockSpec((BN, HW), lambda i: (i, 0)),
            pl.BlockSpec((BN, HW), lambda i: (i, 0)),
            pl.BlockSpec((BN, 1), lambda i: (i, 0)),
            pl.BlockSpec((HW, ZW), lambda i: (0, 0)),
        ],
        out_specs=pl.BlockSpec((BN, ZW), lambda i: (i, 0)),
        out_shape=jax.ShapeDtypeStruct((NP, ZW), jnp.float32),
        input_output_aliases={0: 0},
    )(acc, zg, gel, er, m, s, maskf, spread)


def _final(acc, gb, valid, wro):
    BN = 512

    def body(a_r, gb_r, v_r, w_r, o_r):
        i = pl.program_id(0)
        r = jnp.maximum(a_r[...] + gb_r[...], 0.0)
        hm = (r[:, 0 * HP:1 * HP] + r[:, 1 * HP:2 * HP] + r[:, 2 * HP:3 * HP]
              + r[:, 3 * HP:4 * HP] + r[:, 4 * HP:5 * HP]) * (1.0 / HEADS)
        hm = jnp.where(hm > 0.0, hm, jnp.exp(jnp.minimum(hm, 0.0)) - 1.0)
        hm = hm * v_r[...]
        p = jnp.dot(hm, w_r[...], preferred_element_type=jnp.float32)
        contrib = jnp.sum(p, axis=0, keepdims=True)

        @pl.when(i == 0)
        def _():
            o_r[...] = jnp.zeros_like(o_r)

        o_r[...] += jnp.broadcast_to(contrib, o_r.shape)

    return pl.pallas_call(
        body,
        grid=(NP // BN,),
        in_specs=[
            pl.BlockSpec((BN, ZW), lambda i: (i, 0)),
            pl.BlockSpec((1, ZW), lambda i: (0, 0)),
            pl.BlockSpec((BN, 1), lambda i: (i, 0)),
            pl.BlockSpec((HP, 128), lambda i: (0, 0)),
        ],
        out_specs=pl.BlockSpec((8, 128), lambda i: (0, 0)),
        out_shape=jax.ShapeDtypeStruct((8, 128), jnp.float32),
    )(acc, gb, valid, wro)


def _pack_gate_w(w):
    out = jnp.zeros((HP, G4), jnp.float32)
    for g in range(4):
        out = out.at[:H, g * HP:g * HP + H].set(w[g * H:(g + 1) * H, :].T)
    return out


def _pack_gate_b(b):
    out = jnp.zeros((1, G4), jnp.float32)
    for g in range(4):
        out = out.at[0, g * HP:g * HP + H].set(b[g * H:(g + 1) * H])
    return out


def _pad2(w, r, c):
    out = jnp.zeros((r, c), jnp.float32)
    return out.at[:w.shape[0], :w.shape[1]].set(w)


def kernel(node_tokens, edge_index, emb, sage_Wih, sage_Whh, sage_bih,
           sage_bhh, sage_selfW, sage_selfb, sage_neighW, sage_neighb,
           gat_W, gat_attn_l, gat_attn_r, gat_b, readout_W, readout_b):
    f32 = jnp.float32
    i32 = jnp.int32

    tokens = node_tokens.astype(i32)
    src = edge_index[0].astype(i32)
    dst = edge_index[1].astype(i32)

    order = jnp.argsort(dst)
    src_s = src[order]
    deg = jnp.bincount(dst, length=N_NODES)
    starts = jnp.cumsum(deg) - deg
    dmax = jnp.max(deg)

    padn = NP - N_NODES
    deg_p = jnp.concatenate([deg, jnp.zeros((padn,), deg.dtype)]).astype(i32)
    starts_p = jnp.concatenate([starts,
                                jnp.zeros((padn,), starts.dtype)]).astype(i32)
    tokens_p = jnp.concatenate([tokens, jnp.zeros((padn,), i32)])
    valid = (jnp.arange(NP) < N_NODES).astype(f32)[:, None]
    dummy_srcs = jnp.zeros((8,), i32)

    wih = [_pack_gate_w(sage_Wih[l]) for l in range(3)]
    whh = [_pack_gate_w(sage_Whh[l]) for l in range(3)]
    b4 = [_pack_gate_b(sage_bih[l] + sage_bhh[l]) for l in range(3)]
    wself = [_pad2(sage_selfW[l].T, HP, HP) for l in range(3)]
    wneigh = [_pad2(sage_neighW[l].T, HP, HP) for l in range(3)]
    bcomb = [_pad2((sage_selfb[l] + sage_neighb[l])[None, :], 1, HP)
             for l in range(3)]

    wz = jnp.zeros((HP, ZW), f32)
    al = jnp.zeros((ZW, HW), f32)
    ar = jnp.zeros((ZW, HW), f32)
    gb = jnp.zeros((1, ZW), f32)
    spread = jnp.zeros((HW, ZW), f32)
    for g in range(HEADS):
        wz = wz.at[:H, g * HP:g * HP + H].set(gat_W[:, g * H:(g + 1) * H])
        al = al.at[g * HP:g * HP + H, g].set(gat_attn_l[g])
        ar = ar.at[g * HP:g * HP + H, g].set(gat_attn_r[g])
        gb = gb.at[0, g * HP:g * HP + H].set(gat_b[g * H:(g + 1) * H])
        spread = spread.at[g, g * HP:(g + 1) * HP].set(1.0)
    emb_p = _pad2(emb, 128, HP)
    wro = _pad2(readout_W, HP, 128)

    def pos_at(t, c):
        p = jnp.clip(starts_p + t, 0, N_EDGES - 1)
        return p.reshape(NW, BPW // c, c)

    x = _sc_gather(emb_p, dummy_srcs, tokens_p.reshape(NW, BPW // 32, 32),
                   HP, 32, False)

    for l in range(3):
        xproj = _mm_bias(x, wih[l], b4[l])
        h0 = jnp.zeros((NP, HP), f32)
        c0 = jnp.zeros((NP, HP), f32)

        def lbody(carry, _whh=whh[l], _xp=xproj):
            t, h, c = carry
            xg = _sc_gather(_xp, src_s, pos_at(t, 32), G4, 32, True)
            maskf = (t < deg_p).astype(f32)[:, None]
            h2, c2 = _lstm_step(xg, h, c, _whh, maskf)
            return (t + 1, h2, c2)

        _, hn, _ = lax.while_loop(lambda cr: cr[0] < dmax, lbody,
                                  (jnp.zeros((), i32), h0, c0))
        x = _combine(x, hn, wself[l], wneigh[l], bcomb[l])

    z, el, er = _zel(x, wz, al, ar)

    m0 = jnp.full((NP, HW), -1e30, f32)
    s0 = jnp.zeros((NP, HW), f32)

    def sbody(carry):
        t, m, s = carry
        gel = _sc_gather(el, src_s, pos_at(t, 64), HW, 64, True)
        maskf = (t < deg_p).astype(f32)[:, None]
        m2, s2 = _stats_step(gel, er, m, s, maskf)
        return (t + 1, m2, s2)

    _, m, s = lax.while_loop(lambda cr: cr[0] < dmax, sbody,
                             (jnp.zeros((), i32), m0, s0))

    acc0 = jnp.zeros((NP, ZW), f32)

    def mbody(carry):
        t, acc = carry
        gel = _sc_gather(el, src_s, pos_at(t, 64), HW, 64, True)
        zg = _sc_gather(z, src_s, pos_at(t, 16), ZW, 16, True)
        maskf = (t < deg_p).astype(f32)[:, None]
        acc2 = _msg_step(acc, zg, gel, er, m, s, maskf, spread)
        return (t + 1, acc2)

    _, acc = lax.while_loop(lambda cr: cr[0] < dmax, mbody,
                            (jnp.zeros((), i32), acc0))

    out = _final(acc, gb, valid, wro)
    total = out[0, 0] + f32(N_NODES) * readout_b[0]
    return total.reshape(1)


# speedup vs baseline: 1.6127x; 1.0178x over previous
import functools

import jax
import jax.numpy as jnp
from jax import lax
from jax.experimental import pallas as pl
from jax.experimental.pallas import tpu as pltpu
from jax.experimental.pallas import tpu_sc as plsc

N_NODES = 10000
N_EDGES = 160000
VOCAB = 119
H = 500
HEADS = 5
NEG_SLOPE = 0.2

NP = 10240
HP = 512
G4 = 4 * HP
ZW = HEADS * HP
HW = 128

NW = 32
BPW = NP // NW


def _sc_gather(table, srcs, pos, D, C, two_level, K=1):
    nch = BPW // C
    mesh = plsc.VectorSubcoreMesh(core_axis_name="c", subcore_axis_name="s")

    @functools.partial(
        pl.kernel,
        mesh=mesh,
        out_type=jax.ShapeDtypeStruct((K * NP, D), jnp.float32),
        scratch_types=[
            pltpu.VMEM((K * nch, C), jnp.int32),
            pltpu.VMEM((C,), jnp.int32),
            pltpu.VMEM((C, D), jnp.float32),
            pltpu.SemaphoreType.DMA,
        ],
    )
    def k(table_hbm, srcs_hbm, pos_hbm, out_hbm, pos_v, idx_v, rows_v, sem):
        wid = lax.axis_index("s") * 2 + lax.axis_index("c")
        base = wid * BPW
        pltpu.sync_copy(pos_hbm.at[wid], pos_v)

        def body(q, carry):
            kk = q // nch
            j = q % nch
            if two_level:
                pltpu.async_copy(srcs_hbm.at[pos_v.at[q]], idx_v, sem).wait()
                pltpu.async_copy(table_hbm.at[idx_v], rows_v, sem).wait()
            else:
                pltpu.async_copy(table_hbm.at[pos_v.at[q]], rows_v, sem).wait()
            pltpu.sync_copy(rows_v,
                            out_hbm.at[pl.ds(kk * NP + base + j * C, C)])
            return carry

        lax.fori_loop(0, K * nch, body, 0)

    return k(table, srcs, pos)


def _mm_bias(x, w, b):
    BN = 512
    K, M = w.shape

    def body(x_r, w_r, b_r, o_r):
        o_r[...] = jnp.dot(x_r[...], w_r[...],
                           preferred_element_type=jnp.float32) + b_r[...]

    return pl.pallas_call(
        body,
        grid=(NP // BN,),
        in_specs=[
            pl.BlockSpec((BN, K), lambda i: (i, 0)),
            pl.BlockSpec((K, M), lambda i: (0, 0)),
            pl.BlockSpec((1, M), lambda i: (0, 0)),
        ],
        out_specs=pl.BlockSpec((BN, M), lambda i: (i, 0)),
        out_shape=jax.ShapeDtypeStruct((NP, M), jnp.float32),
    )(x, w, b)


def _lstm_step(xg, h, c, whh, maskf, koff=0):
    BN = 512
    ob = koff * (NP // BN)

    def body(xg_r, h_r, c_r, w_r, m_r, h2_r, c2_r):
        gates = xg_r[...] + jnp.dot(h_r[...].astype(jnp.bfloat16), w_r[...],
                                    preferred_element_type=jnp.float32)
        gi = jax.nn.sigmoid(gates[:, 0 * HP:1 * HP])
        gf = jax.nn.sigmoid(gates[:, 1 * HP:2 * HP])
        gg = jnp.tanh(gates[:, 2 * HP:3 * HP])
        go = jax.nn.sigmoid(gates[:, 3 * HP:4 * HP])
        c_new = gf * c_r[...] + gi * gg
        h_new = go * jnp.tanh(c_new)
        sel = m_r[...] > 0.5
        c2_r[...] = jnp.where(sel, c_new, c_r[...])
        h2_r[...] = jnp.where(sel, h_new, h_r[...])

    return pl.pallas_call(
        body,
        grid=(NP // BN,),
        in_specs=[
            pl.BlockSpec((BN, G4), lambda i: (i + ob, 0)),
            pl.BlockSpec((BN, HP), lambda i: (i, 0)),
            pl.BlockSpec((BN, HP), lambda i: (i, 0)),
            pl.BlockSpec((HP, G4), lambda i: (0, 0)),
            pl.BlockSpec((BN, 1), lambda i: (i, 0)),
        ],
        out_specs=[
            pl.BlockSpec((BN, HP), lambda i: (i, 0)),
            pl.BlockSpec((BN, HP), lambda i: (i, 0)),
        ],
        out_shape=[
            jax.ShapeDtypeStruct((NP, HP), jnp.float32),
            jax.ShapeDtypeStruct((NP, HP), jnp.float32),
        ],
    )(xg, h, c, whh, maskf)


def _combine(x, hn, wself, wneigh, b):
    BN = 512

    def body(x_r, h_r, ws_r, wn_r, b_r, o_r):
        acc = jnp.dot(x_r[...], ws_r[...], preferred_element_type=jnp.float32)
        acc = acc + jnp.dot(h_r[...], wn_r[...],
                            preferred_element_type=jnp.float32)
        o_r[...] = jnp.maximum(acc + b_r[...], 0.0)

    return pl.pallas_call(
        body,
        grid=(NP // BN,),
        in_specs=[
            pl.BlockSpec((BN, HP), lambda i: (i, 0)),
            pl.BlockSpec((BN, HP), lambda i: (i, 0)),
            pl.BlockSpec((HP, HP), lambda i: (0, 0)),
            pl.BlockSpec((HP, HP), lambda i: (0, 0)),
            pl.BlockSpec((1, HP), lambda i: (0, 0)),
        ],
        out_specs=pl.BlockSpec((BN, HP), lambda i: (i, 0)),
        out_shape=jax.ShapeDtypeStruct((NP, HP), jnp.float32),
    )(x, hn, wself, wneigh, b)


def _zel(x, wz, al, ar):
    BN = 512

    def body(x_r, wz_r, al_r, ar_r, z_r, el_r, er_r):
        z = jnp.dot(x_r[...], wz_r[...], preferred_element_type=jnp.float32)
        z_r[...] = z
        el_r[...] = jnp.dot(z, al_r[...], preferred_element_type=jnp.float32)
        er_r[...] = jnp.dot(z, ar_r[...], preferred_element_type=jnp.float32)

    return pl.pallas_call(
        body,
        grid=(NP // BN,),
        in_specs=[
            pl.BlockSpec((BN, HP), lambda i: (i, 0)),
            pl.BlockSpec((HP, ZW), lambda i: (0, 0)),
            pl.BlockSpec((ZW, HW), lambda i: (0, 0)),
            pl.BlockSpec((ZW, HW), lambda i: (0, 0)),
        ],
        out_specs=[
            pl.BlockSpec((BN, ZW), lambda i: (i, 0)),
            pl.BlockSpec((BN, HW), lambda i: (i, 0)),
            pl.BlockSpec((BN, HW), lambda i: (i, 0)),
        ],
        out_shape=[
            jax.ShapeDtypeStruct((NP, ZW), jnp.float32),
            jax.ShapeDtypeStruct((NP, HW), jnp.float32),
            jax.ShapeDtypeStruct((NP, HW), jnp.float32),
        ],
    )(x, wz, al, ar)


def _leaky(v):
    return jnp.where(v >= 0.0, v, NEG_SLOPE * v)


def _stats_step(gel, er, m, s, maskf, koff=0):
    def body(gel_r, er_r, m_r, s_r, mk_r, m2_r, s2_r):
        e = _leaky(gel_r[...] + er_r[...])
        m_old = m_r[...]
        m_new = jnp.maximum(m_old, e)
        s_new = s_r[...] * jnp.exp(m_old - m_new) + jnp.exp(e - m_new)
        sel = mk_r[...] > 0.5
        m2_r[...] = jnp.where(sel, m_new, m_old)
        s2_r[...] = jnp.where(sel, s_new, s_r[...])

    BN = 2048
    ob = koff * (NP // BN)
    return pl.pallas_call(
        body,
        grid=(NP // BN,),
        in_specs=[pl.BlockSpec((BN, HW), lambda i: (i + ob, 0))]
        + [pl.BlockSpec((BN, HW), lambda i: (i, 0))] * 3
        + [pl.BlockSpec((BN, 1), lambda i: (i, 0))],
        out_specs=[pl.BlockSpec((BN, HW), lambda i: (i, 0))] * 2,
        out_shape=[jax.ShapeDtypeStruct((NP, HW), jnp.float32)] * 2,
    )(gel, er, m, s, maskf)


def _msg_step(acc, zg, gel, er, m, s, maskf, spread, koff=0):
    BN = 512
    ob = koff * (NP // BN)

    def body(a_r, zg_r, gel_r, er_r, m_r, s_r, mk_r, sp_r, o_r):
        e = _leaky(gel_r[...] + er_r[...])
        alpha = jnp.exp(e - m_r[...]) / jnp.maximum(s_r[...], 1e-9)
        alpha = jnp.where(mk_r[...] > 0.5, alpha, 0.0)
        aw = jnp.dot(alpha, sp_r[...], preferred_element_type=jnp.float32)
        o_r[...] = a_r[...] + aw * zg_r[...]

    return pl.pallas_call(
        body,
        grid=(NP // BN,),
        in_specs=[
            pl.BlockSpec((BN, ZW), lambda i: (i, 0)),
            pl.BlockSpec((BN, ZW), lambda i: (i + ob, 0)),
            pl.BlockSpec((BN, HW), lambda i: (i + ob, 0)),
            pl.BlockSpec((BN, HW), lambda i: (i, 0)),
            pl.BlockSpec((BN, HW), lambda i: (i, 0)),
            pl.BlockSpec((BN, HW), lambda i: (i, 0)),
            pl.BlockSpec((BN, 1), lambda i: (i, 0)),
            pl.BlockSpec((HW, ZW), lambda i: (0, 0)),
        ],
        out_specs=pl.BlockSpec((BN, ZW), lambda i: (i, 0)),
        out_shape=jax.ShapeDtypeStruct((NP, ZW), jnp.float32),
        input_output_aliases={0: 0},
    )(acc, zg, gel, er, m, s, maskf, spread)


def _final(acc, gb, valid, wro):
    BN = 512

    def body(a_r, gb_r, v_r, w_r, o_r):
        i = pl.program_id(0)
        r = jnp.maximum(a_r[...] + gb_r[...], 0.0)
        hm = (r[:, 0 * HP:1 * HP] + r[:, 1 * HP:2 * HP] + r[:, 2 * HP:3 * HP]
              + r[:, 3 * HP:4 * HP] + r[:, 4 * HP:5 * HP]) * (1.0 / HEADS)
        hm = jnp.where(hm > 0.0, hm, jnp.exp(jnp.minimum(hm, 0.0)) - 1.0)
        hm = hm * v_r[...]
        p = jnp.dot(hm, w_r[...], preferred_element_type=jnp.float32)
        contrib = jnp.sum(p, axis=0, keepdims=True)

        @pl.when(i == 0)
        def _():
            o_r[...] = jnp.zeros_like(o_r)

        o_r[...] += jnp.broadcast_to(contrib, o_r.shape)

    return pl.pallas_call(
        body,
        grid=(NP // BN,),
        in_specs=[
            pl.BlockSpec((BN, ZW), lambda i: (i, 0)),
            pl.BlockSpec((1, ZW), lambda i: (0, 0)),
            pl.BlockSpec((BN, 1), lambda i: (i, 0)),
            pl.BlockSpec((HP, 128), lambda i: (0, 0)),
        ],
        out_specs=pl.BlockSpec((8, 128), lambda i: (0, 0)),
        out_shape=jax.ShapeDtypeStruct((8, 128), jnp.float32),
    )(acc, gb, valid, wro)


def _pack_gate_w(w):
    out = jnp.zeros((HP, G4), jnp.float32)
    for g in range(4):
        out = out.at[:H, g * HP:g * HP + H].set(w[g * H:(g + 1) * H, :].T)
    return out


def _pack_gate_b(b):
    out = jnp.zeros((1, G4), jnp.float32)
    for g in range(4):
        out = out.at[0, g * HP:g * HP + H].set(b[g * H:(g + 1) * H])
    return out


def _pad2(w, r, c):
    out = jnp.zeros((r, c), jnp.float32)
    return out.at[:w.shape[0], :w.shape[1]].set(w)


def kernel(node_tokens, edge_index, emb, sage_Wih, sage_Whh, sage_bih,
           sage_bhh, sage_selfW, sage_selfb, sage_neighW, sage_neighb,
           gat_W, gat_attn_l, gat_attn_r, gat_b, readout_W, readout_b):
    f32 = jnp.float32
    i32 = jnp.int32

    tokens = node_tokens.astype(i32)
    src = edge_index[0].astype(i32)
    dst = edge_index[1].astype(i32)

    order = jnp.argsort(dst)
    src_s = src[order]
    deg = jnp.bincount(dst, length=N_NODES)
    starts = jnp.cumsum(deg) - deg
    dmax = jnp.max(deg)

    padn = NP - N_NODES
    deg_p = jnp.concatenate([deg, jnp.zeros((padn,), deg.dtype)]).astype(i32)
    starts_p = jnp.concatenate([starts,
                                jnp.zeros((padn,), starts.dtype)]).astype(i32)
    tokens_p = jnp.concatenate([tokens, jnp.zeros((padn,), i32)])
    valid = (jnp.arange(NP) < N_NODES).astype(f32)[:, None]
    dummy_srcs = jnp.zeros((8,), i32)

    wih = [_pack_gate_w(sage_Wih[l]) for l in range(3)]
    whh = [_pack_gate_w(sage_Whh[l]) for l in range(3)]
    b4 = [_pack_gate_b(sage_bih[l] + sage_bhh[l]) for l in range(3)]
    wself = [_pad2(sage_selfW[l].T, HP, HP) for l in range(3)]
    wneigh = [_pad2(sage_neighW[l].T, HP, HP) for l in range(3)]
    bcomb = [_pad2((sage_selfb[l] + sage_neighb[l])[None, :], 1, HP)
             for l in range(3)]

    wz = jnp.zeros((HP, ZW), f32)
    al = jnp.zeros((ZW, HW), f32)
    ar = jnp.zeros((ZW, HW), f32)
    gb = jnp.zeros((1, ZW), f32)
    spread = jnp.zeros((HW, ZW), f32)
    for g in range(HEADS):
        wz = wz.at[:H, g * HP:g * HP + H].set(gat_W[:, g * H:(g + 1) * H])
        al = al.at[g * HP:g * HP + H, g].set(gat_attn_l[g])
        ar = ar.at[g * HP:g * HP + H, g].set(gat_attn_r[g])
        gb = gb.at[0, g * HP:g * HP + H].set(gat_b[g * H:(g + 1) * H])
        spread = spread.at[g, g * HP:(g + 1) * HP].set(1.0)
    emb_p = _pad2(emb, 128, HP)
    wro = _pad2(readout_W, HP, 128)

    def pos_at(t, c, kb=1):
        p = jnp.clip(starts_p[None, :] + t + jnp.arange(kb, dtype=i32)[:, None],
                     0, N_EDGES - 1)
        p = p.reshape(kb, NW, BPW // c, c)
        return jnp.transpose(p, (1, 0, 2, 3)).reshape(NW, kb * (BPW // c), c)

    x = _sc_gather(emb_p, dummy_srcs, tokens_p.reshape(NW, BPW // 32, 32),
                   HP, 32, False)

    K = 8
    for l in range(3):
        xproj = _mm_bias(x, wih[l], b4[l])
        h0 = jnp.zeros((NP, HP), f32)
        c0 = jnp.zeros((NP, HP), f32)
        whhb = whh[l].astype(jnp.bfloat16)

        def lbody(carry, _whh=whhb, _xp=xproj):
            t, h, c = carry
            xg = _sc_gather(_xp, src_s, pos_at(t, 32, K), G4, 32, True, K)
            for k in range(K):
                maskf = (t + k < deg_p).astype(f32)[:, None]
                h, c = _lstm_step(xg, h, c, _whh, maskf, koff=k)
            return (t + K, h, c)

        _, hn, _ = lax.while_loop(lambda cr: cr[0] < dmax, lbody,
                                  (jnp.zeros((), i32), h0, c0))
        x = _combine(x, hn, wself[l], wneigh[l], bcomb[l])

    z, el, er = _zel(x, wz, al, ar)

    m0 = jnp.full((NP, HW), -1e30, f32)
    s0 = jnp.zeros((NP, HW), f32)

    def sbody(carry):
        t, m, s = carry
        gel = _sc_gather(el, src_s, pos_at(t, 64, K), HW, 64, True, K)
        for k in range(K):
            maskf = (t + k < deg_p).astype(f32)[:, None]
            m, s = _stats_step(gel, er, m, s, maskf, koff=k)
        return (t + K, m, s)

    _, m, s = lax.while_loop(lambda cr: cr[0] < dmax, sbody,
                             (jnp.zeros((), i32), m0, s0))

    acc0 = jnp.zeros((NP, ZW), f32)

    def mbody(carry):
        t, acc = carry
        gel = _sc_gather(el, src_s, pos_at(t, 64, K), HW, 64, True, K)
        zg = _sc_gather(z, src_s, pos_at(t, 16, K), ZW, 16, True, K)
        for k in range(K):
            maskf = (t + k < deg_p).astype(f32)[:, None]
            acc = _msg_step(acc, zg, gel, er, m, s, maskf, spread, koff=k)
        return (t + K, acc)

    _, acc = lax.while_loop(lambda cr: cr[0] < dmax, mbody,
                            (jnp.zeros((), i32), acc0))

    out = _final(acc, gb, valid, wro)
    total = out[0, 0] + f32(N_NODES) * readout_b[0]
    return total.reshape(1)


# speedup vs baseline: 1.6977x; 1.0527x over previous
import functools

import jax
import jax.numpy as jnp
from jax import lax
from jax.experimental import pallas as pl
from jax.experimental.pallas import tpu as pltpu
from jax.experimental.pallas import tpu_sc as plsc

N_NODES = 10000
N_EDGES = 160000
VOCAB = 119
H = 500
HEADS = 5
NEG_SLOPE = 0.2

NP = 10240
HP = 512
G4 = 4 * HP
ZW = HEADS * HP
HW = 128

NW = 32
BPW = NP // NW


def _sc_gather(table, srcs, pos, D, C, two_level, K=1):
    nch = BPW // C
    Q = K * nch
    pipelined = (Q * C) % 128 == 0 and Q % 2 == 0
    NG = (Q * C) // 128 if pipelined else Q
    mesh = plsc.VectorSubcoreMesh(core_axis_name="c", subcore_axis_name="s")

    def out_off(q):
        return (q // nch) * NP + (q % nch) * C

    @functools.partial(
        pl.kernel,
        mesh=mesh,
        out_type=jax.ShapeDtypeStruct((K * NP, D), jnp.float32),
        scratch_types=[
            pltpu.VMEM((NG, 128) if pipelined else (Q, C), jnp.int32),
            pltpu.VMEM((NG, 128) if pipelined else (Q, C), jnp.int32),
            pltpu.VMEM((C, D), jnp.float32),
            pltpu.VMEM((C, D), jnp.float32),
            pltpu.SemaphoreType.DMA,
            pltpu.SemaphoreType.DMA,
            pltpu.SemaphoreType.DMA,
            pltpu.SemaphoreType.DMA,
            pltpu.SemaphoreType.DMA,
        ],
    )
    def k(table_hbm, srcs_hbm, pos_hbm, out_hbm, pos_v, idx_v,
          rows0, rows1, sem_i, sem_r0, sem_r1, sem_w0, sem_w1):
        wid = lax.axis_index("s") * 2 + lax.axis_index("c")
        base = wid * BPW
        pltpu.sync_copy(pos_hbm.at[wid], pos_v)
        if not pipelined:
            def sbody(q, carry):
                src_ref = pos_v.at[q]
                if two_level:
                    pltpu.async_copy(srcs_hbm.at[src_ref], idx_v.at[q],
                                     sem_i).wait()
                    src_ref = idx_v.at[q]
                pltpu.async_copy(table_hbm.at[src_ref], rows0, sem_r0).wait()
                pltpu.sync_copy(rows0, out_hbm.at[pl.ds(base + out_off(q), C)])
                return carry

            lax.fori_loop(0, Q, sbody, 0)
            return
        if two_level:
            for g in range(NG):
                pltpu.make_async_copy(srcs_hbm.at[pos_v.at[g]],
                                      idx_v.at[g], sem_i).start()
            for g in range(NG):
                pltpu.make_async_copy(srcs_hbm.at[pos_v.at[g]],
                                      idx_v.at[g], sem_i).wait()
            idx_ref = idx_v
        else:
            idx_ref = pos_v

        def chunk_idx(q):
            flat = q * C
            return idx_ref.at[flat // 128, pl.ds(flat % 128, C)]

        def wr(q, rows, sem):
            return pltpu.make_async_copy(
                rows, out_hbm.at[pl.ds(base + out_off(q), C)], sem)

        def body(r, carry):
            q0 = 2 * r
            q1 = 2 * r + 1

            @pl.when(r >= 1)
            def _():
                wr(q0 - 2, rows0, sem_w0).wait()

            pltpu.make_async_copy(table_hbm.at[chunk_idx(q0)], rows0,
                                  sem_r0).start()

            @pl.when(r >= 1)
            def _():
                wr(q1 - 2, rows1, sem_w1).wait()

            pltpu.make_async_copy(table_hbm.at[chunk_idx(q1)], rows1,
                                  sem_r1).start()
            pltpu.make_async_copy(table_hbm.at[chunk_idx(q0)], rows0,
                                  sem_r0).wait()
            wr(q0, rows0, sem_w0).start()
            pltpu.make_async_copy(table_hbm.at[chunk_idx(q1)], rows1,
                                  sem_r1).wait()
            wr(q1, rows1, sem_w1).start()
            return carry

        lax.fori_loop(0, Q // 2, body, 0)
        wr(Q - 2, rows0, sem_w0).wait()
        wr(Q - 1, rows1, sem_w1).wait()

    return k(table, srcs, pos.reshape(NW, NG, 128) if pipelined else pos)


def _mm_bias(x, w, b):
    BN = 512
    K, M = w.shape

    def body(x_r, w_r, b_r, o_r):
        o_r[...] = jnp.dot(x_r[...], w_r[...],
                           preferred_element_type=jnp.float32) + b_r[...]

    return pl.pallas_call(
        body,
        grid=(NP // BN,),
        in_specs=[
            pl.BlockSpec((BN, K), lambda i: (i, 0)),
            pl.BlockSpec((K, M), lambda i: (0, 0)),
            pl.BlockSpec((1, M), lambda i: (0, 0)),
        ],
        out_specs=pl.BlockSpec((BN, M), lambda i: (i, 0)),
        out_shape=jax.ShapeDtypeStruct((NP, M), jnp.float32),
    )(x, w, b)


def _lstm_step(xg, h, c, whh, maskf, koff=0):
    BN = 512
    ob = koff * (NP // BN)

    def body(xg_r, h_r, c_r, w_r, m_r, h2_r, c2_r):
        gates = xg_r[...] + jnp.dot(h_r[...].astype(jnp.bfloat16), w_r[...],
                                    preferred_element_type=jnp.float32)
        gi = jax.nn.sigmoid(gates[:, 0 * HP:1 * HP])
        gf = jax.nn.sigmoid(gates[:, 1 * HP:2 * HP])
        gg = jnp.tanh(gates[:, 2 * HP:3 * HP])
        go = jax.nn.sigmoid(gates[:, 3 * HP:4 * HP])
        c_new = gf * c_r[...] + gi * gg
        h_new = go * jnp.tanh(c_new)
        sel = m_r[...] > 0.5
        c2_r[...] = jnp.where(sel, c_new, c_r[...])
        h2_r[...] = jnp.where(sel, h_new, h_r[...])

    return pl.pallas_call(
        body,
        grid=(NP // BN,),
        in_specs=[
            pl.BlockSpec((BN, G4), lambda i: (i + ob, 0)),
            pl.BlockSpec((BN, HP), lambda i: (i, 0)),
            pl.BlockSpec((BN, HP), lambda i: (i, 0)),
            pl.BlockSpec((HP, G4), lambda i: (0, 0)),
            pl.BlockSpec((BN, 1), lambda i: (i, 0)),
        ],
        out_specs=[
            pl.BlockSpec((BN, HP), lambda i: (i, 0)),
            pl.BlockSpec((BN, HP), lambda i: (i, 0)),
        ],
        out_shape=[
            jax.ShapeDtypeStruct((NP, HP), jnp.float32),
            jax.ShapeDtypeStruct((NP, HP), jnp.float32),
        ],
    )(xg, h, c, whh, maskf)


def _combine(x, hn, wself, wneigh, b):
    BN = 512

    def body(x_r, h_r, ws_r, wn_r, b_r, o_r):
        acc = jnp.dot(x_r[...], ws_r[...], preferred_element_type=jnp.float32)
        acc = acc + jnp.dot(h_r[...], wn_r[...],
                            preferred_element_type=jnp.float32)
        o_r[...] = jnp.maximum(acc + b_r[...], 0.0)

    return pl.pallas_call(
        body,
        grid=(NP // BN,),
        in_specs=[
            pl.BlockSpec((BN, HP), lambda i: (i, 0)),
            pl.BlockSpec((BN, HP), lambda i: (i, 0)),
            pl.BlockSpec((HP, HP), lambda i: (0, 0)),
            pl.BlockSpec((HP, HP), lambda i: (0, 0)),
            pl.BlockSpec((1, HP), lambda i: (0, 0)),
        ],
        out_specs=pl.BlockSpec((BN, HP), lambda i: (i, 0)),
        out_shape=jax.ShapeDtypeStruct((NP, HP), jnp.float32),
    )(x, hn, wself, wneigh, b)


def _zel(x, wz, al, ar):
    BN = 512

    def body(x_r, wz_r, al_r, ar_r, z_r, el_r, er_r):
        z = jnp.dot(x_r[...], wz_r[...], preferred_element_type=jnp.float32)
        z_r[...] = z
        el_r[...] = jnp.dot(z, al_r[...], preferred_element_type=jnp.float32)
        er_r[...] = jnp.dot(z, ar_r[...], preferred_element_type=jnp.float32)

    return pl.pallas_call(
        body,
        grid=(NP // BN,),
        in_specs=[
            pl.BlockSpec((BN, HP), lambda i: (i, 0)),
            pl.BlockSpec((HP, ZW), lambda i: (0, 0)),
            pl.BlockSpec((ZW, HW), lambda i: (0, 0)),
            pl.BlockSpec((ZW, HW), lambda i: (0, 0)),
        ],
        out_specs=[
            pl.BlockSpec((BN, ZW), lambda i: (i, 0)),
            pl.BlockSpec((BN, HW), lambda i: (i, 0)),
            pl.BlockSpec((BN, HW), lambda i: (i, 0)),
        ],
        out_shape=[
            jax.ShapeDtypeStruct((NP, ZW), jnp.float32),
            jax.ShapeDtypeStruct((NP, HW), jnp.float32),
            jax.ShapeDtypeStruct((NP, HW), jnp.float32),
        ],
    )(x, wz, al, ar)


def _leaky(v):
    return jnp.where(v >= 0.0, v, NEG_SLOPE * v)


def _stats_step(gel, er, m, s, maskf, koff=0):
    def body(gel_r, er_r, m_r, s_r, mk_r, m2_r, s2_r):
        e = _leaky(gel_r[...] + er_r[...])
        m_old = m_r[...]
        m_new = jnp.maximum(m_old, e)
        s_new = s_r[...] * jnp.exp(m_old - m_new) + jnp.exp(e - m_new)
        sel = mk_r[...] > 0.5
        m2_r[...] = jnp.where(sel, m_new, m_old)
        s2_r[...] = jnp.where(sel, s_new, s_r[...])

    BN = 2048
    ob = koff * (NP // BN)
    return pl.pallas_call(
        body,
        grid=(NP // BN,),
        in_specs=[pl.BlockSpec((BN, HW), lambda i: (i + ob, 0))]
        + [pl.BlockSpec((BN, HW), lambda i: (i, 0))] * 3
        + [pl.BlockSpec((BN, 1), lambda i: (i, 0))],
        out_specs=[pl.BlockSpec((BN, HW), lambda i: (i, 0))] * 2,
        out_shape=[jax.ShapeDtypeStruct((NP, HW), jnp.float32)] * 2,
    )(gel, er, m, s, maskf)


def _msg_step(acc, zg, gel, er, m, s, maskf, spread, koff=0):
    BN = 512
    ob = koff * (NP // BN)

    def body(a_r, zg_r, gel_r, er_r, m_r, s_r, mk_r, sp_r, o_r):
        e = _leaky(gel_r[...] + er_r[...])
        alpha = jnp.exp(e - m_r[...]) / jnp.maximum(s_r[...], 1e-9)
        alpha = jnp.where(mk_r[...] > 0.5, alpha, 0.0)
        aw = jnp.dot(alpha, sp_r[...], preferred_element_type=jnp.float32)
        o_r[...] = a_r[...] + aw * zg_r[...]

    return pl.pallas_call(
        body,
        grid=(NP // BN,),
        in_specs=[
            pl.BlockSpec((BN, ZW), lambda i: (i, 0)),
            pl.BlockSpec((BN, ZW), lambda i: (i + ob, 0)),
            pl.BlockSpec((BN, HW), lambda i: (i + ob, 0)),
            pl.BlockSpec((BN, HW), lambda i: (i, 0)),
            pl.BlockSpec((BN, HW), lambda i: (i, 0)),
            pl.BlockSpec((BN, HW), lambda i: (i, 0)),
            pl.BlockSpec((BN, 1), lambda i: (i, 0)),
            pl.BlockSpec((HW, ZW), lambda i: (0, 0)),
        ],
        out_specs=pl.BlockSpec((BN, ZW), lambda i: (i, 0)),
        out_shape=jax.ShapeDtypeStruct((NP, ZW), jnp.float32),
        input_output_aliases={0: 0},
    )(acc, zg, gel, er, m, s, maskf, spread)


def _final(acc, gb, valid, wro):
    BN = 512

    def body(a_r, gb_r, v_r, w_r, o_r):
        i = pl.program_id(0)
        r = jnp.maximum(a_r[...] + gb_r[...], 0.0)
        hm = (r[:, 0 * HP:1 * HP] + r[:, 1 * HP:2 * HP] + r[:, 2 * HP:3 * HP]
              + r[:, 3 * HP:4 * HP] + r[:, 4 * HP:5 * HP]) * (1.0 / HEADS)
        hm = jnp.where(hm > 0.0, hm, jnp.exp(jnp.minimum(hm, 0.0)) - 1.0)
        hm = hm * v_r[...]
        p = jnp.dot(hm, w_r[...], preferred_element_type=jnp.float32)
        contrib = jnp.sum(p, axis=0, keepdims=True)

        @pl.when(i == 0)
        def _():
            o_r[...] = jnp.zeros_like(o_r)

        o_r[...] += jnp.broadcast_to(contrib, o_r.shape)

    return pl.pallas_call(
        body,
        grid=(NP // BN,),
        in_specs=[
            pl.BlockSpec((BN, ZW), lambda i: (i, 0)),
            pl.BlockSpec((1, ZW), lambda i: (0, 0)),
            pl.BlockSpec((BN, 1), lambda i: (i, 0)),
            pl.BlockSpec((HP, 128), lambda i: (0, 0)),
        ],
        out_specs=pl.BlockSpec((8, 128), lambda i: (0, 0)),
        out_shape=jax.ShapeDtypeStruct((8, 128), jnp.float32),
    )(acc, gb, valid, wro)


def _pack_gate_w(w):
    out = jnp.zeros((HP, G4), jnp.float32)
    for g in range(4):
        out = out.at[:H, g * HP:g * HP + H].set(w[g * H:(g + 1) * H, :].T)
    return out


def _pack_gate_b(b):
    out = jnp.zeros((1, G4), jnp.float32)
    for g in range(4):
        out = out.at[0, g * HP:g * HP + H].set(b[g * H:(g + 1) * H])
    return out


def _pad2(w, r, c):
    out = jnp.zeros((r, c), jnp.float32)
    return out.at[:w.shape[0], :w.shape[1]].set(w)


def kernel(node_tokens, edge_index, emb, sage_Wih, sage_Whh, sage_bih,
           sage_bhh, sage_selfW, sage_selfb, sage_neighW, sage_neighb,
           gat_W, gat_attn_l, gat_attn_r, gat_b, readout_W, readout_b):
    f32 = jnp.float32
    i32 = jnp.int32

    tokens = node_tokens.astype(i32)
    src = edge_index[0].astype(i32)
    dst = edge_index[1].astype(i32)

    order = jnp.argsort(dst)
    src_s = src[order]
    deg = jnp.bincount(dst, length=N_NODES)
    starts = jnp.cumsum(deg) - deg
    dmax = jnp.max(deg)

    padn = NP - N_NODES
    deg_p = jnp.concatenate([deg, jnp.zeros((padn,), deg.dtype)]).astype(i32)
    starts_p = jnp.concatenate([starts,
                                jnp.zeros((padn,), starts.dtype)]).astype(i32)
    tokens_p = jnp.concatenate([tokens, jnp.zeros((padn,), i32)])
    valid = (jnp.arange(NP) < N_NODES).astype(f32)[:, None]
    dummy_srcs = jnp.zeros((8,), i32)

    wih = [_pack_gate_w(sage_Wih[l]) for l in range(3)]
    whh = [_pack_gate_w(sage_Whh[l]) for l in range(3)]
    b4 = [_pack_gate_b(sage_bih[l] + sage_bhh[l]) for l in range(3)]
    wself = [_pad2(sage_selfW[l].T, HP, HP) for l in range(3)]
    wneigh = [_pad2(sage_neighW[l].T, HP, HP) for l in range(3)]
    bcomb = [_pad2((sage_selfb[l] + sage_neighb[l])[None, :], 1, HP)
             for l in range(3)]

    wz = jnp.zeros((HP, ZW), f32)
    al = jnp.zeros((ZW, HW), f32)
    ar = jnp.zeros((ZW, HW), f32)
    gb = jnp.zeros((1, ZW), f32)
    spread = jnp.zeros((HW, ZW), f32)
    for g in range(HEADS):
        wz = wz.at[:H, g * HP:g * HP + H].set(gat_W[:, g * H:(g + 1) * H])
        al = al.at[g * HP:g * HP + H, g].set(gat_attn_l[g])
        ar = ar.at[g * HP:g * HP + H, g].set(gat_attn_r[g])
        gb = gb.at[0, g * HP:g * HP + H].set(gat_b[g * H:(g + 1) * H])
        spread = spread.at[g, g * HP:(g + 1) * HP].set(1.0)
    emb_p = _pad2(emb, 128, HP)
    wro = _pad2(readout_W, HP, 128)

    def pos_at(t, c, kb=1):
        p = jnp.clip(starts_p[None, :] + t + jnp.arange(kb, dtype=i32)[:, None],
                     0, N_EDGES - 1)
        p = p.reshape(kb, NW, BPW // c, c)
        return jnp.transpose(p, (1, 0, 2, 3)).reshape(NW, kb * (BPW // c), c)

    x = _sc_gather(emb_p, dummy_srcs, tokens_p.reshape(NW, BPW // 32, 32),
                   HP, 32, False)

    K = 8
    for l in range(3):
        xproj = _mm_bias(x, wih[l], b4[l])
        h0 = jnp.zeros((NP, HP), f32)
        c0 = jnp.zeros((NP, HP), f32)
        whhb = whh[l].astype(jnp.bfloat16)

        def lbody(carry, _whh=whhb, _xp=xproj):
            t, h, c = carry
            xg = _sc_gather(_xp, src_s, pos_at(t, 16, K), G4, 16, True, K)
            for k in range(K):
                maskf = (t + k < deg_p).astype(f32)[:, None]
                h, c = _lstm_step(xg, h, c, _whh, maskf, koff=k)
            return (t + K, h, c)

        _, hn, _ = lax.while_loop(lambda cr: cr[0] < dmax, lbody,
                                  (jnp.zeros((), i32), h0, c0))
        x = _combine(x, hn, wself[l], wneigh[l], bcomb[l])

    z, el, er = _zel(x, wz, al, ar)

    m0 = jnp.full((NP, HW), -1e30, f32)
    s0 = jnp.zeros((NP, HW), f32)

    def sbody(carry):
        t, m, s = carry
        gel = _sc_gather(el, src_s, pos_at(t, 64, K), HW, 64, True, K)
        for k in range(K):
            maskf = (t + k < deg_p).astype(f32)[:, None]
            m, s = _stats_step(gel, er, m, s, maskf, koff=k)
        return (t + K, m, s)

    _, m, s = lax.while_loop(lambda cr: cr[0] < dmax, sbody,
                             (jnp.zeros((), i32), m0, s0))

    acc0 = jnp.zeros((NP, ZW), f32)

    def mbody(carry):
        t, acc = carry
        gel = _sc_gather(el, src_s, pos_at(t, 64, K), HW, 64, True, K)
        zg = _sc_gather(z, src_s, pos_at(t, 16, K), ZW, 16, True, K)
        for k in range(K):
            maskf = (t + k < deg_p).astype(f32)[:, None]
            acc = _msg_step(acc, zg, gel, er, m, s, maskf, spread, koff=k)
        return (t + K, acc)

    _, acc = lax.while_loop(lambda cr: cr[0] < dmax, mbody,
                            (jnp.zeros((), i32), acc0))

    out = _final(acc, gb, valid, wro)
    total = out[0, 0] + f32(N_NODES) * readout_b[0]
    return total.reshape(1)


# speedup vs baseline: 2.2531x; 1.3271x over previous
import functools

import jax
import jax.numpy as jnp
from jax import lax
from jax.experimental import pallas as pl
from jax.experimental.pallas import tpu as pltpu
from jax.experimental.pallas import tpu_sc as plsc

N_NODES = 10000
N_EDGES = 160000
VOCAB = 119
H = 500
HEADS = 5
NEG_SLOPE = 0.2

NP = 10240
HP = 512
G4 = 4 * HP
ZW = HEADS * HP
HW = 128

NW = 32
BPW = NP // NW


def _sc_gather(table, srcs, pos, D, C, two_level, K=1):
    nch = BPW // C
    Q = K * nch
    pipelined = (Q * C) % 128 == 0 and Q % 2 == 0
    NG = (Q * C) // 128 if pipelined else Q
    row_shape = (C, D)
    out_sds = (K * NP, D)
    mesh = plsc.VectorSubcoreMesh(core_axis_name="c", subcore_axis_name="s")

    def out_off(q):
        return (q // nch) * NP + (q % nch) * C

    @functools.partial(
        pl.kernel,
        mesh=mesh,
        out_type=jax.ShapeDtypeStruct(out_sds, jnp.float32),
        scratch_types=[
            pltpu.VMEM((NG, 128) if pipelined else (Q, C), jnp.int32),
            pltpu.VMEM((NG, 128) if pipelined else (Q, C), jnp.int32),
            pltpu.VMEM(row_shape, jnp.float32),
            pltpu.VMEM(row_shape, jnp.float32),
            pltpu.SemaphoreType.DMA,
            pltpu.SemaphoreType.DMA,
            pltpu.SemaphoreType.DMA,
            pltpu.SemaphoreType.DMA,
            pltpu.SemaphoreType.DMA,
        ],
    )
    def k(table_hbm, srcs_hbm, pos_hbm, out_hbm, pos_v, idx_v,
          rows0, rows1, sem_i, sem_r0, sem_r1, sem_w0, sem_w1):
        wid = lax.axis_index("s") * 2 + lax.axis_index("c")
        base = wid * BPW
        pltpu.sync_copy(pos_hbm.at[wid], pos_v)
        if not pipelined:
            def sbody(q, carry):
                src_ref = pos_v.at[q]
                if two_level:
                    pltpu.async_copy(srcs_hbm.at[src_ref], idx_v.at[q],
                                     sem_i).wait()
                    src_ref = idx_v.at[q]
                pltpu.async_copy(table_hbm.at[src_ref], rows0, sem_r0).wait()
                pltpu.sync_copy(rows0, out_hbm.at[pl.ds(base + out_off(q), C)])
                return carry

            lax.fori_loop(0, Q, sbody, 0)
            return
        if two_level:
            for g in range(NG):
                pltpu.make_async_copy(srcs_hbm.at[pos_v.at[g]],
                                      idx_v.at[g], sem_i).start()
            for g in range(NG):
                pltpu.make_async_copy(srcs_hbm.at[pos_v.at[g]],
                                      idx_v.at[g], sem_i).wait()
            idx_ref = idx_v
        else:
            idx_ref = pos_v

        def chunk_idx(q):
            flat = q * C
            return idx_ref.at[flat // 128, pl.ds(flat % 128, C)]

        def wr(q, rows, sem):
            return pltpu.make_async_copy(
                rows, out_hbm.at[pl.ds(base + out_off(q), C)], sem)

        def body(r, carry):
            q0 = 2 * r
            q1 = 2 * r + 1

            @pl.when(r >= 1)
            def _():
                wr(q0 - 2, rows0, sem_w0).wait()

            pltpu.make_async_copy(table_hbm.at[chunk_idx(q0)], rows0,
                                  sem_r0).start()

            @pl.when(r >= 1)
            def _():
                wr(q1 - 2, rows1, sem_w1).wait()

            pltpu.make_async_copy(table_hbm.at[chunk_idx(q1)], rows1,
                                  sem_r1).start()
            pltpu.make_async_copy(table_hbm.at[chunk_idx(q0)], rows0,
                                  sem_r0).wait()
            wr(q0, rows0, sem_w0).start()
            pltpu.make_async_copy(table_hbm.at[chunk_idx(q1)], rows1,
                                  sem_r1).wait()
            wr(q1, rows1, sem_w1).start()
            return carry

        lax.fori_loop(0, Q // 2, body, 0)
        wr(Q - 2, rows0, sem_w0).wait()
        wr(Q - 1, rows1, sem_w1).wait()

    return k(table, srcs, pos.reshape(NW, NG, 128) if pipelined else pos)


def _pack_pairs(a):
    w2 = a.shape[1] // 2
    ab = a.astype(jnp.bfloat16)
    hi = jax.lax.bitcast_convert_type(ab[:, :w2], jnp.uint16).astype(jnp.uint32)
    lo = jax.lax.bitcast_convert_type(ab[:, w2:], jnp.uint16).astype(jnp.uint32)
    return jax.lax.bitcast_convert_type((hi << 16) | lo, jnp.float32)


def _unpack_hi(x):
    xi = jax.lax.bitcast_convert_type(x, jnp.int32)
    return jax.lax.bitcast_convert_type(xi & jnp.int32(-65536), jnp.float32)


def _unpack_lo(x):
    xi = jax.lax.bitcast_convert_type(x, jnp.int32)
    return jax.lax.bitcast_convert_type(xi << 16, jnp.float32)


def _mm_bias(x, w, b):
    BN = 512
    K, M = w.shape

    def body(x_r, w_r, b_r, o_r):
        o_r[...] = jnp.dot(x_r[...], w_r[...],
                           preferred_element_type=jnp.float32) + b_r[...]

    return pl.pallas_call(
        body,
        grid=(NP // BN,),
        in_specs=[
            pl.BlockSpec((BN, K), lambda i: (i, 0)),
            pl.BlockSpec((K, M), lambda i: (0, 0)),
            pl.BlockSpec((1, M), lambda i: (0, 0)),
        ],
        out_specs=pl.BlockSpec((BN, M), lambda i: (i, 0)),
        out_shape=jax.ShapeDtypeStruct((NP, M), jnp.float32),
    )(x, w, b)


def _lstm_step(xg, h, c, whh, maskf, koff=0):
    BN = 512
    ob = koff * (NP // BN)

    def body(xg_r, h_r, c_r, w_r, m_r, h2_r, c2_r):
        d = jnp.dot(h_r[...].astype(jnp.bfloat16), w_r[...],
                    preferred_element_type=jnp.float32)
        xhi = _unpack_hi(xg_r[...])
        xlo = _unpack_lo(xg_r[...])
        gi = jax.nn.sigmoid(xhi[:, 0 * HP:1 * HP] + d[:, 0 * HP:1 * HP])
        gf = jax.nn.sigmoid(xhi[:, 1 * HP:2 * HP] + d[:, 1 * HP:2 * HP])
        gg = jnp.tanh(xlo[:, 0 * HP:1 * HP] + d[:, 2 * HP:3 * HP])
        go = jax.nn.sigmoid(xlo[:, 1 * HP:2 * HP] + d[:, 3 * HP:4 * HP])
        c_new = gf * c_r[...] + gi * gg
        h_new = go * jnp.tanh(c_new)
        sel = m_r[...] > 0.5
        c2_r[...] = jnp.where(sel, c_new, c_r[...])
        h2_r[...] = jnp.where(sel, h_new, h_r[...])

    return pl.pallas_call(
        body,
        grid=(NP // BN,),
        in_specs=[
            pl.BlockSpec((BN, G4 // 2), lambda i: (i + ob, 0)),
            pl.BlockSpec((BN, HP), lambda i: (i, 0)),
            pl.BlockSpec((BN, HP), lambda i: (i, 0)),
            pl.BlockSpec((HP, G4), lambda i: (0, 0)),
            pl.BlockSpec((BN, 1), lambda i: (i, 0)),
        ],
        out_specs=[
            pl.BlockSpec((BN, HP), lambda i: (i, 0)),
            pl.BlockSpec((BN, HP), lambda i: (i, 0)),
        ],
        out_shape=[
            jax.ShapeDtypeStruct((NP, HP), jnp.float32),
            jax.ShapeDtypeStruct((NP, HP), jnp.float32),
        ],
    )(xg, h, c, whh, maskf)


def _combine(x, hn, wself, wneigh, b):
    BN = 512

    def body(x_r, h_r, ws_r, wn_r, b_r, o_r):
        acc = jnp.dot(x_r[...], ws_r[...], preferred_element_type=jnp.float32)
        acc = acc + jnp.dot(h_r[...], wn_r[...],
                            preferred_element_type=jnp.float32)
        o_r[...] = jnp.maximum(acc + b_r[...], 0.0)

    return pl.pallas_call(
        body,
        grid=(NP // BN,),
        in_specs=[
            pl.BlockSpec((BN, HP), lambda i: (i, 0)),
            pl.BlockSpec((BN, HP), lambda i: (i, 0)),
            pl.BlockSpec((HP, HP), lambda i: (0, 0)),
            pl.BlockSpec((HP, HP), lambda i: (0, 0)),
            pl.BlockSpec((1, HP), lambda i: (0, 0)),
        ],
        out_specs=pl.BlockSpec((BN, HP), lambda i: (i, 0)),
        out_shape=jax.ShapeDtypeStruct((NP, HP), jnp.float32),
    )(x, hn, wself, wneigh, b)


def _zel(x, wz, al, ar):
    BN = 512

    def body(x_r, wz_r, al_r, ar_r, z_r, el_r, er_r):
        z = jnp.dot(x_r[...], wz_r[...], preferred_element_type=jnp.float32)
        z_r[...] = z
        el_r[...] = jnp.dot(z, al_r[...], preferred_element_type=jnp.float32)
        er_r[...] = jnp.dot(z, ar_r[...], preferred_element_type=jnp.float32)

    return pl.pallas_call(
        body,
        grid=(NP // BN,),
        in_specs=[
            pl.BlockSpec((BN, HP), lambda i: (i, 0)),
            pl.BlockSpec((HP, ZW), lambda i: (0, 0)),
            pl.BlockSpec((ZW, HW), lambda i: (0, 0)),
            pl.BlockSpec((ZW, HW), lambda i: (0, 0)),
        ],
        out_specs=[
            pl.BlockSpec((BN, ZW), lambda i: (i, 0)),
            pl.BlockSpec((BN, HW), lambda i: (i, 0)),
            pl.BlockSpec((BN, HW), lambda i: (i, 0)),
        ],
        out_shape=[
            jax.ShapeDtypeStruct((NP, ZW), jnp.float32),
            jax.ShapeDtypeStruct((NP, HW), jnp.float32),
            jax.ShapeDtypeStruct((NP, HW), jnp.float32),
        ],
    )(x, wz, al, ar)


def _leaky(v):
    return jnp.where(v >= 0.0, v, NEG_SLOPE * v)


def _stats_step(gel, er, m, s, maskf, koff=0):
    def body(gel_r, er_r, m_r, s_r, mk_r, m2_r, s2_r):
        e = _leaky(gel_r[...] + er_r[...])
        m_old = m_r[...]
        m_new = jnp.maximum(m_old, e)
        s_new = s_r[...] * jnp.exp(m_old - m_new) + jnp.exp(e - m_new)
        sel = mk_r[...] > 0.5
        m2_r[...] = jnp.where(sel, m_new, m_old)
        s2_r[...] = jnp.where(sel, s_new, s_r[...])

    BN = 2048
    ob = koff * (NP // BN)
    return pl.pallas_call(
        body,
        grid=(NP // BN,),
        in_specs=[pl.BlockSpec((BN, HW), lambda i: (i + ob, 0))]
        + [pl.BlockSpec((BN, HW), lambda i: (i, 0))] * 3
        + [pl.BlockSpec((BN, 1), lambda i: (i, 0))],
        out_specs=[pl.BlockSpec((BN, HW), lambda i: (i, 0))] * 2,
        out_shape=[jax.ShapeDtypeStruct((NP, HW), jnp.float32)] * 2,
    )(gel, er, m, s, maskf)


def _msg_step(acc, zg, gel, er, m, s, maskf, spread, koff=0):
    BN = 512
    ob = koff * (NP // BN)

    def body(a_r, zg_r, gel_r, er_r, m_r, s_r, mk_r, sp_r, o_r):
        e = _leaky(gel_r[...] + er_r[...])
        alpha = jnp.exp(e - m_r[...]) / jnp.maximum(s_r[...], 1e-9)
        alpha = jnp.where(mk_r[...] > 0.5, alpha, 0.0)
        aw = jnp.dot(alpha, sp_r[...], preferred_element_type=jnp.float32)
        zfull = jnp.concatenate([_unpack_hi(zg_r[...]),
                                 _unpack_lo(zg_r[...])], axis=1)
        o_r[...] = a_r[...] + aw * zfull

    return pl.pallas_call(
        body,
        grid=(NP // BN,),
        in_specs=[
            pl.BlockSpec((BN, ZW), lambda i: (i, 0)),
            pl.BlockSpec((BN, ZW // 2), lambda i: (i + ob, 0)),
            pl.BlockSpec((BN, HW), lambda i: (i + ob, 0)),
            pl.BlockSpec((BN, HW), lambda i: (i, 0)),
            pl.BlockSpec((BN, HW), lambda i: (i, 0)),
            pl.BlockSpec((BN, HW), lambda i: (i, 0)),
            pl.BlockSpec((BN, 1), lambda i: (i, 0)),
            pl.BlockSpec((HW, ZW), lambda i: (0, 0)),
        ],
        out_specs=pl.BlockSpec((BN, ZW), lambda i: (i, 0)),
        out_shape=jax.ShapeDtypeStruct((NP, ZW), jnp.float32),
        input_output_aliases={0: 0},
    )(acc, zg, gel, er, m, s, maskf, spread)


def _final(acc, gb, valid, wro):
    BN = 512

    def body(a_r, gb_r, v_r, w_r, o_r):
        i = pl.program_id(0)
        r = jnp.maximum(a_r[...] + gb_r[...], 0.0)
        hm = (r[:, 0 * HP:1 * HP] + r[:, 1 * HP:2 * HP] + r[:, 2 * HP:3 * HP]
              + r[:, 3 * HP:4 * HP] + r[:, 4 * HP:5 * HP]) * (1.0 / HEADS)
        hm = jnp.where(hm > 0.0, hm, jnp.exp(jnp.minimum(hm, 0.0)) - 1.0)
        hm = hm * v_r[...]
        p = jnp.dot(hm, w_r[...], preferred_element_type=jnp.float32)
        contrib = jnp.sum(p, axis=0, keepdims=True)

        @pl.when(i == 0)
        def _():
            o_r[...] = jnp.zeros_like(o_r)

        o_r[...] += jnp.broadcast_to(contrib, o_r.shape)

    return pl.pallas_call(
        body,
        grid=(NP // BN,),
        in_specs=[
            pl.BlockSpec((BN, ZW), lambda i: (i, 0)),
            pl.BlockSpec((1, ZW), lambda i: (0, 0)),
            pl.BlockSpec((BN, 1), lambda i: (i, 0)),
            pl.BlockSpec((HP, 128), lambda i: (0, 0)),
        ],
        out_specs=pl.BlockSpec((8, 128), lambda i: (0, 0)),
        out_shape=jax.ShapeDtypeStruct((8, 128), jnp.float32),
    )(acc, gb, valid, wro)


def _pack_gate_w(w):
    out = jnp.zeros((HP, G4), jnp.float32)
    for g in range(4):
        out = out.at[:H, g * HP:g * HP + H].set(w[g * H:(g + 1) * H, :].T)
    return out


def _pack_gate_b(b):
    out = jnp.zeros((1, G4), jnp.float32)
    for g in range(4):
        out = out.at[0, g * HP:g * HP + H].set(b[g * H:(g + 1) * H])
    return out


def _pad2(w, r, c):
    out = jnp.zeros((r, c), jnp.float32)
    return out.at[:w.shape[0], :w.shape[1]].set(w)


def kernel(node_tokens, edge_index, emb, sage_Wih, sage_Whh, sage_bih,
           sage_bhh, sage_selfW, sage_selfb, sage_neighW, sage_neighb,
           gat_W, gat_attn_l, gat_attn_r, gat_b, readout_W, readout_b):
    f32 = jnp.float32
    i32 = jnp.int32

    tokens = node_tokens.astype(i32)
    src = edge_index[0].astype(i32)
    dst = edge_index[1].astype(i32)

    order = jnp.argsort(dst)
    src_s = src[order]
    deg = jnp.bincount(dst, length=N_NODES)
    starts = jnp.cumsum(deg) - deg
    dmax = jnp.max(deg)

    padn = NP - N_NODES
    deg_p = jnp.concatenate([deg, jnp.zeros((padn,), deg.dtype)]).astype(i32)
    starts_p = jnp.concatenate([starts,
                                jnp.zeros((padn,), starts.dtype)]).astype(i32)
    tokens_p = jnp.concatenate([tokens, jnp.zeros((padn,), i32)])
    valid = (jnp.arange(NP) < N_NODES).astype(f32)[:, None]
    dummy_srcs = jnp.zeros((8,), i32)

    wih = [_pack_gate_w(sage_Wih[l]) for l in range(3)]
    whh = [_pack_gate_w(sage_Whh[l]) for l in range(3)]
    b4 = [_pack_gate_b(sage_bih[l] + sage_bhh[l]) for l in range(3)]
    wself = [_pad2(sage_selfW[l].T, HP, HP) for l in range(3)]
    wneigh = [_pad2(sage_neighW[l].T, HP, HP) for l in range(3)]
    bcomb = [_pad2((sage_selfb[l] + sage_neighb[l])[None, :], 1, HP)
             for l in range(3)]

    wz = jnp.zeros((HP, ZW), f32)
    al = jnp.zeros((ZW, HW), f32)
    ar = jnp.zeros((ZW, HW), f32)
    gb = jnp.zeros((1, ZW), f32)
    spread = jnp.zeros((HW, ZW), f32)
    for g in range(HEADS):
        wz = wz.at[:H, g * HP:g * HP + H].set(gat_W[:, g * H:(g + 1) * H])
        al = al.at[g * HP:g * HP + H, g].set(gat_attn_l[g])
        ar = ar.at[g * HP:g * HP + H, g].set(gat_attn_r[g])
        gb = gb.at[0, g * HP:g * HP + H].set(gat_b[g * H:(g + 1) * H])
        spread = spread.at[g, g * HP:(g + 1) * HP].set(1.0)
    emb_p = _pad2(emb, 128, HP)
    wro = _pad2(readout_W, HP, 128)

    def pos_at(t, c, kb=1):
        p = jnp.clip(starts_p[None, :] + t + jnp.arange(kb, dtype=i32)[:, None],
                     0, N_EDGES - 1)
        p = p.reshape(kb, NW, BPW // c, c)
        return jnp.transpose(p, (1, 0, 2, 3)).reshape(NW, kb * (BPW // c), c)

    x = _sc_gather(emb_p, dummy_srcs, tokens_p.reshape(NW, BPW // 32, 32),
                   HP, 32, False)

    K = 8
    for l in range(3):
        xproj = _mm_bias(x, wih[l], b4[l])
        xpb = _pack_pairs(xproj)
        h0 = jnp.zeros((NP, HP), f32)
        c0 = jnp.zeros((NP, HP), f32)
        whhb = whh[l].astype(jnp.bfloat16)

        def lbody(carry, _whh=whhb, _xp=xpb):
            t, h, c = carry
            xg = _sc_gather(_xp, src_s, pos_at(t, 32, K), G4 // 2, 32,
                            True, K)
            for k in range(K):
                maskf = (t + k < deg_p).astype(f32)[:, None]
                h, c = _lstm_step(xg, h, c, _whh, maskf, koff=k)
            return (t + K, h, c)

        _, hn, _ = lax.while_loop(lambda cr: cr[0] < dmax, lbody,
                                  (jnp.zeros((), i32), h0, c0))
        x = _combine(x, hn, wself[l], wneigh[l], bcomb[l])

    z, el, er = _zel(x, wz, al, ar)

    m0 = jnp.full((NP, HW), -1e30, f32)
    s0 = jnp.zeros((NP, HW), f32)

    def sbody(carry):
        t, m, s = carry
        gel = _sc_gather(el, src_s, pos_at(t, 64, K), HW, 64, True, K)
        for k in range(K):
            maskf = (t + k < deg_p).astype(f32)[:, None]
            m, s = _stats_step(gel, er, m, s, maskf, koff=k)
        return (t + K, m, s)

    _, m, s = lax.while_loop(lambda cr: cr[0] < dmax, sbody,
                             (jnp.zeros((), i32), m0, s0))

    acc0 = jnp.zeros((NP, ZW), f32)

    zpk = _pack_pairs(z)

    def mbody(carry):
        t, acc = carry
        gel = _sc_gather(el, src_s, pos_at(t, 64, K), HW, 64, True, K)
        zg = _sc_gather(zpk, src_s, pos_at(t, 32, K), ZW // 2, 32, True, K)
        for k in range(K):
            maskf = (t + k < deg_p).astype(f32)[:, None]
            acc = _msg_step(acc, zg, gel, er, m, s, maskf, spread, koff=k)
        return (t + K, acc)

    _, acc = lax.while_loop(lambda cr: cr[0] < dmax, mbody,
                            (jnp.zeros((), i32), acc0))

    out = _final(acc, gb, valid, wro)
    total = out[0, 0] + f32(N_NODES) * readout_b[0]
    return total.reshape(1)


# speedup vs baseline: 2.3033x; 1.0223x over previous
import functools

import jax
import jax.numpy as jnp
from jax import lax
from jax.experimental import pallas as pl
from jax.experimental.pallas import tpu as pltpu
from jax.experimental.pallas import tpu_sc as plsc

N_NODES = 10000
N_EDGES = 160000
VOCAB = 119
H = 500
HEADS = 5
NEG_SLOPE = 0.2

NP = 10240
HP = 512
G4 = 4 * HP
ZW = HEADS * HP
HW = 128

NW = 32
BPW = NP // NW


def _sc_gather(table, srcs, pos, D, C, two_level, K=1):
    nch = BPW // C
    Q = K * nch
    pipelined = (Q * C) % 128 == 0 and Q % 2 == 0
    NG = (Q * C) // 128 if pipelined else Q
    row_shape = (C, D)
    out_sds = (K * NP, D)
    mesh = plsc.VectorSubcoreMesh(core_axis_name="c", subcore_axis_name="s")

    def out_off(q):
        return (q // nch) * NP + (q % nch) * C

    @functools.partial(
        pl.kernel,
        mesh=mesh,
        out_type=jax.ShapeDtypeStruct(out_sds, jnp.float32),
        scratch_types=[
            pltpu.VMEM((NG, 128) if pipelined else (Q, C), jnp.int32),
            pltpu.VMEM((NG, 128) if pipelined else (Q, C), jnp.int32),
            pltpu.VMEM(row_shape, jnp.float32),
            pltpu.VMEM(row_shape, jnp.float32),
            pltpu.SemaphoreType.DMA,
            pltpu.SemaphoreType.DMA,
            pltpu.SemaphoreType.DMA,
            pltpu.SemaphoreType.DMA,
            pltpu.SemaphoreType.DMA,
        ],
    )
    def k(table_hbm, srcs_hbm, pos_hbm, out_hbm, pos_v, idx_v,
          rows0, rows1, sem_i, sem_r0, sem_r1, sem_w0, sem_w1):
        wid = lax.axis_index("s") * 2 + lax.axis_index("c")
        base = wid * BPW
        pltpu.sync_copy(pos_hbm.at[wid], pos_v)
        if not pipelined:
            def sbody(q, carry):
                src_ref = pos_v.at[q]
                if two_level:
                    pltpu.async_copy(srcs_hbm.at[src_ref], idx_v.at[q],
                                     sem_i).wait()
                    src_ref = idx_v.at[q]
                pltpu.async_copy(table_hbm.at[src_ref], rows0, sem_r0).wait()
                pltpu.sync_copy(rows0, out_hbm.at[pl.ds(base + out_off(q), C)])
                return carry

            lax.fori_loop(0, Q, sbody, 0)
            return
        if two_level:
            for g in range(NG):
                pltpu.make_async_copy(srcs_hbm.at[pos_v.at[g]],
                                      idx_v.at[g], sem_i).start()
            for g in range(NG):
                pltpu.make_async_copy(srcs_hbm.at[pos_v.at[g]],
                                      idx_v.at[g], sem_i).wait()
            idx_ref = idx_v
        else:
            idx_ref = pos_v

        def chunk_idx(q):
            flat = q * C
            return idx_ref.at[flat // 128, pl.ds(flat % 128, C)]

        def wr(q, rows, sem):
            return pltpu.make_async_copy(
                rows, out_hbm.at[pl.ds(base + out_off(q), C)], sem)

        def body(r, carry):
            q0 = 2 * r
            q1 = 2 * r + 1

            @pl.when(r >= 1)
            def _():
                wr(q0 - 2, rows0, sem_w0).wait()

            pltpu.make_async_copy(table_hbm.at[chunk_idx(q0)], rows0,
                                  sem_r0).start()

            @pl.when(r >= 1)
            def _():
                wr(q1 - 2, rows1, sem_w1).wait()

            pltpu.make_async_copy(table_hbm.at[chunk_idx(q1)], rows1,
                                  sem_r1).start()
            pltpu.make_async_copy(table_hbm.at[chunk_idx(q0)], rows0,
                                  sem_r0).wait()
            wr(q0, rows0, sem_w0).start()
            pltpu.make_async_copy(table_hbm.at[chunk_idx(q1)], rows1,
                                  sem_r1).wait()
            wr(q1, rows1, sem_w1).start()
            return carry

        lax.fori_loop(0, Q // 2, body, 0)
        wr(Q - 2, rows0, sem_w0).wait()
        wr(Q - 1, rows1, sem_w1).wait()

    return k(table, srcs, pos.reshape(NW, NG, 128) if pipelined else pos)


def _pack_pairs(a):
    w2 = a.shape[1] // 2
    ab = a.astype(jnp.bfloat16)
    hi = jax.lax.bitcast_convert_type(ab[:, :w2], jnp.uint16).astype(jnp.uint32)
    lo = jax.lax.bitcast_convert_type(ab[:, w2:], jnp.uint16).astype(jnp.uint32)
    return jax.lax.bitcast_convert_type((hi << 16) | lo, jnp.float32)


def _unpack_hi(x):
    xi = jax.lax.bitcast_convert_type(x, jnp.int32)
    return jax.lax.bitcast_convert_type(xi & jnp.int32(-65536), jnp.float32)


def _unpack_lo(x):
    xi = jax.lax.bitcast_convert_type(x, jnp.int32)
    return jax.lax.bitcast_convert_type(xi << 16, jnp.float32)


def _mm_bias(x, w, b):
    BN = 512
    K, M = w.shape

    def body(x_r, w_r, b_r, o_r):
        o_r[...] = jnp.dot(x_r[...], w_r[...],
                           preferred_element_type=jnp.float32) + b_r[...]

    return pl.pallas_call(
        body,
        grid=(NP // BN,),
        in_specs=[
            pl.BlockSpec((BN, K), lambda i: (i, 0)),
            pl.BlockSpec((K, M), lambda i: (0, 0)),
            pl.BlockSpec((1, M), lambda i: (0, 0)),
        ],
        out_specs=pl.BlockSpec((BN, M), lambda i: (i, 0)),
        out_shape=jax.ShapeDtypeStruct((NP, M), jnp.float32),
    )(x, w, b)


def _lstm_step(xg, h, c, whh, maskf, koff=0):
    BN = 512
    ob = koff * (NP // BN)

    def body(xg_r, h_r, c_r, w_r, m_r, h2_r, c2_r):
        d = jnp.dot(h_r[...].astype(jnp.bfloat16), w_r[...],
                    preferred_element_type=jnp.float32)
        xhi = _unpack_hi(xg_r[...])
        xlo = _unpack_lo(xg_r[...])
        gi = jax.nn.sigmoid(xhi[:, 0 * HP:1 * HP] + d[:, 0 * HP:1 * HP])
        gf = jax.nn.sigmoid(xhi[:, 1 * HP:2 * HP] + d[:, 1 * HP:2 * HP])
        gg = jnp.tanh(xlo[:, 0 * HP:1 * HP] + d[:, 2 * HP:3 * HP])
        go = jax.nn.sigmoid(xlo[:, 1 * HP:2 * HP] + d[:, 3 * HP:4 * HP])
        c_new = gf * c_r[...] + gi * gg
        h_new = go * jnp.tanh(c_new)
        sel = m_r[...] > 0.5
        c2_r[...] = jnp.where(sel, c_new, c_r[...])
        h2_r[...] = jnp.where(sel, h_new, h_r[...])

    return pl.pallas_call(
        body,
        grid=(NP // BN,),
        in_specs=[
            pl.BlockSpec((BN, G4 // 2), lambda i: (i + ob, 0)),
            pl.BlockSpec((BN, HP), lambda i: (i, 0)),
            pl.BlockSpec((BN, HP), lambda i: (i, 0)),
            pl.BlockSpec((HP, G4), lambda i: (0, 0)),
            pl.BlockSpec((BN, 1), lambda i: (i, 0)),
        ],
        out_specs=[
            pl.BlockSpec((BN, HP), lambda i: (i, 0)),
            pl.BlockSpec((BN, HP), lambda i: (i, 0)),
        ],
        out_shape=[
            jax.ShapeDtypeStruct((NP, HP), jnp.float32),
            jax.ShapeDtypeStruct((NP, HP), jnp.float32),
        ],
    )(xg, h, c, whh, maskf)


def _combine(x, hn, wself, wneigh, b):
    BN = 512

    def body(x_r, h_r, ws_r, wn_r, b_r, o_r):
        acc = jnp.dot(x_r[...], ws_r[...], preferred_element_type=jnp.float32)
        acc = acc + jnp.dot(h_r[...], wn_r[...],
                            preferred_element_type=jnp.float32)
        o_r[...] = jnp.maximum(acc + b_r[...], 0.0)

    return pl.pallas_call(
        body,
        grid=(NP // BN,),
        in_specs=[
            pl.BlockSpec((BN, HP), lambda i: (i, 0)),
            pl.BlockSpec((BN, HP), lambda i: (i, 0)),
            pl.BlockSpec((HP, HP), lambda i: (0, 0)),
            pl.BlockSpec((HP, HP), lambda i: (0, 0)),
            pl.BlockSpec((1, HP), lambda i: (0, 0)),
        ],
        out_specs=pl.BlockSpec((BN, HP), lambda i: (i, 0)),
        out_shape=jax.ShapeDtypeStruct((NP, HP), jnp.float32),
    )(x, hn, wself, wneigh, b)


def _zel(x, wz, al, ar):
    BN = 512

    def body(x_r, wz_r, al_r, ar_r, z_r, el_r, er_r):
        z = jnp.dot(x_r[...], wz_r[...], preferred_element_type=jnp.float32)
        z_r[...] = z
        el_r[...] = jnp.dot(z, al_r[...], preferred_element_type=jnp.float32)
        er_r[...] = jnp.dot(z, ar_r[...], preferred_element_type=jnp.float32)

    return pl.pallas_call(
        body,
        grid=(NP // BN,),
        in_specs=[
            pl.BlockSpec((BN, HP), lambda i: (i, 0)),
            pl.BlockSpec((HP, ZW), lambda i: (0, 0)),
            pl.BlockSpec((ZW, HW), lambda i: (0, 0)),
            pl.BlockSpec((ZW, HW), lambda i: (0, 0)),
        ],
        out_specs=[
            pl.BlockSpec((BN, ZW), lambda i: (i, 0)),
            pl.BlockSpec((BN, HW), lambda i: (i, 0)),
            pl.BlockSpec((BN, HW), lambda i: (i, 0)),
        ],
        out_shape=[
            jax.ShapeDtypeStruct((NP, ZW), jnp.float32),
            jax.ShapeDtypeStruct((NP, HW), jnp.float32),
            jax.ShapeDtypeStruct((NP, HW), jnp.float32),
        ],
    )(x, wz, al, ar)


def _leaky(v):
    return jnp.where(v >= 0.0, v, NEG_SLOPE * v)


def _stats_step(gel, er, m, s, maskf, koff=0):
    def body(gel_r, er_r, m_r, s_r, mk_r, m2_r, s2_r):
        e = _leaky(gel_r[...] + er_r[...])
        m_old = m_r[...]
        m_new = jnp.maximum(m_old, e)
        s_new = s_r[...] * jnp.exp(m_old - m_new) + jnp.exp(e - m_new)
        sel = mk_r[...] > 0.5
        m2_r[...] = jnp.where(sel, m_new, m_old)
        s2_r[...] = jnp.where(sel, s_new, s_r[...])

    BN = 2048
    ob = koff * (NP // BN)
    return pl.pallas_call(
        body,
        grid=(NP // BN,),
        in_specs=[pl.BlockSpec((BN, HW), lambda i: (i + ob, 0))]
        + [pl.BlockSpec((BN, HW), lambda i: (i, 0))] * 3
        + [pl.BlockSpec((BN, 1), lambda i: (i, 0))],
        out_specs=[pl.BlockSpec((BN, HW), lambda i: (i, 0))] * 2,
        out_shape=[jax.ShapeDtypeStruct((NP, HW), jnp.float32)] * 2,
    )(gel, er, m, s, maskf)


def _msg_step(acc, zg, er, m, s, maskf, spread, koff=0):
    BN = 512
    ob = koff * (NP // BN)

    def body(a_r, zg_r, er_r, m_r, s_r, mk_r, sp_r, o_r):
        gpk = zg_r[:, ZW // 2:ZW // 2 + HW // 2]
        gel = jnp.concatenate([_unpack_hi(gpk), _unpack_lo(gpk)], axis=1)
        e = _leaky(gel + er_r[...])
        alpha = jnp.exp(e - m_r[...]) / jnp.maximum(s_r[...], 1e-9)
        alpha = jnp.where(mk_r[...] > 0.5, alpha, 0.0)
        aw = jnp.dot(alpha, sp_r[...], preferred_element_type=jnp.float32)
        zfull = jnp.concatenate([_unpack_hi(zg_r[:, :ZW // 2]),
                                 _unpack_lo(zg_r[:, :ZW // 2])], axis=1)
        o_r[...] = a_r[...] + aw * zfull

    return pl.pallas_call(
        body,
        grid=(NP // BN,),
        in_specs=[
            pl.BlockSpec((BN, ZW), lambda i: (i, 0)),
            pl.BlockSpec((BN, ZW // 2 + HW), lambda i: (i + ob, 0)),
            pl.BlockSpec((BN, HW), lambda i: (i, 0)),
            pl.BlockSpec((BN, HW), lambda i: (i, 0)),
            pl.BlockSpec((BN, HW), lambda i: (i, 0)),
            pl.BlockSpec((BN, 1), lambda i: (i, 0)),
            pl.BlockSpec((HW, ZW), lambda i: (0, 0)),
        ],
        out_specs=pl.BlockSpec((BN, ZW), lambda i: (i, 0)),
        out_shape=jax.ShapeDtypeStruct((NP, ZW), jnp.float32),
        input_output_aliases={0: 0},
    )(acc, zg, er, m, s, maskf, spread)


def _final(acc, gb, valid, wro):
    BN = 512

    def body(a_r, gb_r, v_r, w_r, o_r):
        i = pl.program_id(0)
        r = jnp.maximum(a_r[...] + gb_r[...], 0.0)
        hm = (r[:, 0 * HP:1 * HP] + r[:, 1 * HP:2 * HP] + r[:, 2 * HP:3 * HP]
              + r[:, 3 * HP:4 * HP] + r[:, 4 * HP:5 * HP]) * (1.0 / HEADS)
        hm = jnp.where(hm > 0.0, hm, jnp.exp(jnp.minimum(hm, 0.0)) - 1.0)
        hm = hm * v_r[...]
        p = jnp.dot(hm, w_r[...], preferred_element_type=jnp.float32)
        contrib = jnp.sum(p, axis=0, keepdims=True)

        @pl.when(i == 0)
        def _():
            o_r[...] = jnp.zeros_like(o_r)

        o_r[...] += jnp.broadcast_to(contrib, o_r.shape)

    return pl.pallas_call(
        body,
        grid=(NP // BN,),
        in_specs=[
            pl.BlockSpec((BN, ZW), lambda i: (i, 0)),
            pl.BlockSpec((1, ZW), lambda i: (0, 0)),
            pl.BlockSpec((BN, 1), lambda i: (i, 0)),
            pl.BlockSpec((HP, 128), lambda i: (0, 0)),
        ],
        out_specs=pl.BlockSpec((8, 128), lambda i: (0, 0)),
        out_shape=jax.ShapeDtypeStruct((8, 128), jnp.float32),
    )(acc, gb, valid, wro)


def _pack_gate_w(w):
    out = jnp.zeros((HP, G4), jnp.float32)
    for g in range(4):
        out = out.at[:H, g * HP:g * HP + H].set(w[g * H:(g + 1) * H, :].T)
    return out


def _pack_gate_b(b):
    out = jnp.zeros((1, G4), jnp.float32)
    for g in range(4):
        out = out.at[0, g * HP:g * HP + H].set(b[g * H:(g + 1) * H])
    return out


def _pad2(w, r, c):
    out = jnp.zeros((r, c), jnp.float32)
    return out.at[:w.shape[0], :w.shape[1]].set(w)


def kernel(node_tokens, edge_index, emb, sage_Wih, sage_Whh, sage_bih,
           sage_bhh, sage_selfW, sage_selfb, sage_neighW, sage_neighb,
           gat_W, gat_attn_l, gat_attn_r, gat_b, readout_W, readout_b):
    f32 = jnp.float32
    i32 = jnp.int32

    tokens = node_tokens.astype(i32)
    src = edge_index[0].astype(i32)
    dst = edge_index[1].astype(i32)

    order = jnp.argsort(dst)
    src_s = src[order]
    deg = jnp.bincount(dst, length=N_NODES)
    starts = jnp.cumsum(deg) - deg
    dmax = jnp.max(deg)

    padn = NP - N_NODES
    deg_p = jnp.concatenate([deg, jnp.zeros((padn,), deg.dtype)]).astype(i32)
    starts_p = jnp.concatenate([starts,
                                jnp.zeros((padn,), starts.dtype)]).astype(i32)
    tokens_p = jnp.concatenate([tokens, jnp.zeros((padn,), i32)])
    valid = (jnp.arange(NP) < N_NODES).astype(f32)[:, None]
    dummy_srcs = jnp.zeros((8,), i32)

    wih = [_pack_gate_w(sage_Wih[l]) for l in range(3)]
    whh = [_pack_gate_w(sage_Whh[l]) for l in range(3)]
    b4 = [_pack_gate_b(sage_bih[l] + sage_bhh[l]) for l in range(3)]
    wself = [_pad2(sage_selfW[l].T, HP, HP) for l in range(3)]
    wneigh = [_pad2(sage_neighW[l].T, HP, HP) for l in range(3)]
    bcomb = [_pad2((sage_selfb[l] + sage_neighb[l])[None, :], 1, HP)
             for l in range(3)]

    wz = jnp.zeros((HP, ZW), f32)
    al = jnp.zeros((ZW, HW), f32)
    ar = jnp.zeros((ZW, HW), f32)
    gb = jnp.zeros((1, ZW), f32)
    spread = jnp.zeros((HW, ZW), f32)
    for g in range(HEADS):
        wz = wz.at[:H, g * HP:g * HP + H].set(gat_W[:, g * H:(g + 1) * H])
        al = al.at[g * HP:g * HP + H, g].set(gat_attn_l[g])
        ar = ar.at[g * HP:g * HP + H, g].set(gat_attn_r[g])
        gb = gb.at[0, g * HP:g * HP + H].set(gat_b[g * H:(g + 1) * H])
        spread = spread.at[g, g * HP:(g + 1) * HP].set(1.0)
    emb_p = _pad2(emb, 128, HP)
    wro = _pad2(readout_W, HP, 128)

    def pos_at(t, c, kb=1):
        p = jnp.clip(starts_p[None, :] + t + jnp.arange(kb, dtype=i32)[:, None],
                     0, N_EDGES - 1)
        p = p.reshape(kb, NW, BPW // c, c)
        return jnp.transpose(p, (1, 0, 2, 3)).reshape(NW, kb * (BPW // c), c)

    x = _sc_gather(emb_p, dummy_srcs, tokens_p.reshape(NW, BPW // 32, 32),
                   HP, 32, False)

    K = 8
    for l in range(3):
        xproj = _mm_bias(x, wih[l], b4[l])
        xpb = _pack_pairs(xproj)
        h0 = jnp.zeros((NP, HP), f32)
        c0 = jnp.zeros((NP, HP), f32)
        whhb = whh[l].astype(jnp.bfloat16)

        def lbody(carry, _whh=whhb, _xp=xpb):
            t, h, c = carry
            xg = _sc_gather(_xp, src_s, pos_at(t, 32, K), G4 // 2, 32,
                            True, K)
            for k in range(K):
                maskf = (t + k < deg_p).astype(f32)[:, None]
                h, c = _lstm_step(xg, h, c, _whh, maskf, koff=k)
            return (t + K, h, c)

        _, hn, _ = lax.while_loop(lambda cr: cr[0] < dmax, lbody,
                                  (jnp.zeros((), i32), h0, c0))
        x = _combine(x, hn, wself[l], wneigh[l], bcomb[l])

    z, el, er = _zel(x, wz, al, ar)

    m0 = jnp.full((NP, HW), -1e30, f32)
    s0 = jnp.zeros((NP, HW), f32)

    def sbody(carry):
        t, m, s = carry
        gel = _sc_gather(el, src_s, pos_at(t, 64, K), HW, 64, True, K)
        for k in range(K):
            maskf = (t + k < deg_p).astype(f32)[:, None]
            m, s = _stats_step(gel, er, m, s, maskf, koff=k)
        return (t + K, m, s)

    _, m, s = lax.while_loop(lambda cr: cr[0] < dmax, sbody,
                             (jnp.zeros((), i32), m0, s0))

    acc0 = jnp.zeros((NP, ZW), f32)

    zpk = jnp.concatenate([_pack_pairs(z), _pack_pairs(el),
                           jnp.zeros((NP, HW // 2), f32)], axis=1)

    def mbody(carry):
        t, acc = carry
        zg = _sc_gather(zpk, src_s, pos_at(t, 32, K), ZW // 2 + HW, 32,
                        True, K)
        for k in range(K):
            maskf = (t + k < deg_p).astype(f32)[:, None]
            acc = _msg_step(acc, zg, er, m, s, maskf, spread, koff=k)
        return (t + K, acc)

    _, acc = lax.while_loop(lambda cr: cr[0] < dmax, mbody,
                            (jnp.zeros((), i32), acc0))

    out = _final(acc, gb, valid, wro)
    total = out[0, 0] + f32(N_NODES) * readout_b[0]
    return total.reshape(1)
